# Initial kernel scaffold; baseline (speedup 1.0000x reference)
import jax
import jax.numpy as jnp
from jax.experimental import pallas as pl


def kernel(input_ids, word_table, pos_table, gamma, beta):
    raise NotImplementedError("write your pallas kernel here")


# speedup vs baseline: 5.3131x; 5.3131x over previous
import functools

import jax
import jax.numpy as jnp
from jax.experimental import pallas as pl
from jax.experimental.pallas import tpu as pltpu

_EPS = 1e-5
_SEQ_TILE_MAX = 256


def _round_up(x: int, m: int) -> int:
    return (x + m - 1) // m * m


def _gather_ln_kernel(seq_tile, n_waves,
                      ids_ref,
                      word_hbm,
                      pos_ref,
                      gamma_ref,
                      beta_ref,
                      out_ref,
                      tok_buf,
                      sems):
    g = pl.program_id(0)
    base = g * seq_tile
    wave = seq_tile // n_waves

    for i in range(seq_tile):
        rid = ids_ref[base + i]
        pltpu.make_async_copy(word_hbm.at[pl.ds(rid, 1)],
                              tok_buf.at[pl.ds(i, 1)],
                              sems.at[i // wave]).start()

    gamma = gamma_ref[...]
    beta = beta_ref[...]

    for w in range(n_waves):
        rows = pl.ds(w * wave, wave)
        pltpu.make_async_copy(word_hbm.at[pl.ds(0, wave)],
                              tok_buf.at[rows],
                              sems.at[w]).wait()
        z = tok_buf[rows, :] + pos_ref[rows, :]
        mean = jnp.mean(z, axis=-1, keepdims=True)
        c = z - mean
        var = jnp.mean(c * c, axis=-1, keepdims=True)
        out_ref[rows, :] = c * jax.lax.rsqrt(var + _EPS) * gamma + beta


def kernel(input_ids, word_table, pos_table, gamma, beta):
    B, S = input_ids.shape
    V, H = word_table.shape

    seq_tile = min(_round_up(_SEQ_TILE_MAX, 8), _round_up(S, 8))
    s_pad = _round_up(S, seq_tile)
    n_seq = s_pad // seq_tile
    n_waves = 4 if seq_tile % 4 == 0 and (seq_tile // 4) % 8 == 0 else 1

    ids = jnp.clip(input_ids.astype(jnp.int32), 0, V - 1)
    if s_pad != S:
        ids = jnp.pad(ids, ((0, 0), (0, s_pad - S)))
    pos = pos_table[:S].astype(jnp.float32)
    if s_pad != S:
        pos = jnp.pad(pos, ((0, s_pad - S), (0, 0)))

    gamma2 = gamma.reshape(1, H).astype(jnp.float32)
    beta2 = beta.reshape(1, H).astype(jnp.float32)

    grid = (B * n_seq,)
    kernel_fn = functools.partial(_gather_ln_kernel, seq_tile, n_waves)
    out = pl.pallas_call(
        kernel_fn,
        out_shape=jax.ShapeDtypeStruct((B * s_pad, H), jnp.float32),
        grid_spec=pltpu.PrefetchScalarGridSpec(
            num_scalar_prefetch=1,
            grid=grid,
            in_specs=[
                pl.BlockSpec(memory_space=pl.ANY),
                pl.BlockSpec((seq_tile, H), lambda g, *_: (g % n_seq, 0)),
                pl.BlockSpec((1, H), lambda g, *_: (0, 0)),
                pl.BlockSpec((1, H), lambda g, *_: (0, 0)),
            ],
            out_specs=pl.BlockSpec((seq_tile, H), lambda g, *_: (g, 0)),
            scratch_shapes=[
                pltpu.VMEM((seq_tile, H), jnp.float32),
                pltpu.SemaphoreType.DMA((n_waves,)),
            ]),
        compiler_params=pltpu.CompilerParams(
            dimension_semantics=("parallel",),
            disable_bounds_checks=True,
            vmem_limit_bytes=64 << 20),
    )(ids.reshape(-1), word_table.astype(jnp.float32), pos, gamma2, beta2)

    out = out.reshape(B, s_pad, H)
    return out if s_pad == S else out[:, :S, :]


# speedup vs baseline: 5.3301x; 1.0032x over previous
import functools

import jax
import jax.numpy as jnp
from jax.experimental import pallas as pl
from jax.experimental.pallas import tpu as pltpu

_EPS = 1e-5
_SEQ_TILE_MAX = 256


def _round_up(x: int, m: int) -> int:
    return (x + m - 1) // m * m


def _gather_ln_kernel(seq_tile, n_waves,
                      ids_ref,
                      word_hbm,
                      pos_ref,
                      gamma_ref,
                      beta_ref,
                      out_ref,
                      tok_buf,
                      sems):
    g = pl.program_id(0)
    base = g * seq_tile
    wave = seq_tile // n_waves

    for i in range(seq_tile):
        rid = ids_ref[base + i]
        pltpu.make_async_copy(word_hbm.at[pl.ds(rid, 1)],
                              tok_buf.at[pl.ds(i, 1)],
                              sems.at[i // wave]).start(priority=i & 1)

    gamma = gamma_ref[...]
    beta = beta_ref[...]

    for w in range(n_waves):
        rows = pl.ds(w * wave, wave)
        pltpu.make_async_copy(word_hbm.at[pl.ds(0, wave)],
                              tok_buf.at[rows],
                              sems.at[w]).wait()
        z = tok_buf[rows, :] + pos_ref[rows, :]
        mean = jnp.mean(z, axis=-1, keepdims=True)
        c = z - mean
        var = jnp.mean(c * c, axis=-1, keepdims=True)
        out_ref[rows, :] = c * jax.lax.rsqrt(var + _EPS) * gamma + beta


def kernel(input_ids, word_table, pos_table, gamma, beta):
    B, S = input_ids.shape
    V, H = word_table.shape

    seq_tile = min(_round_up(_SEQ_TILE_MAX, 8), _round_up(S, 8))
    s_pad = _round_up(S, seq_tile)
    n_seq = s_pad // seq_tile
    n_waves = 4 if seq_tile % 4 == 0 and (seq_tile // 4) % 8 == 0 else 1

    ids = jnp.clip(input_ids.astype(jnp.int32), 0, V - 1)
    if s_pad != S:
        ids = jnp.pad(ids, ((0, 0), (0, s_pad - S)))
    pos = pos_table[:S].astype(jnp.float32)
    if s_pad != S:
        pos = jnp.pad(pos, ((0, s_pad - S), (0, 0)))

    gamma2 = gamma.reshape(1, H).astype(jnp.float32)
    beta2 = beta.reshape(1, H).astype(jnp.float32)

    grid = (B * n_seq,)
    kernel_fn = functools.partial(_gather_ln_kernel, seq_tile, n_waves)
    out = pl.pallas_call(
        kernel_fn,
        out_shape=jax.ShapeDtypeStruct((B * s_pad, H), jnp.float32),
        grid_spec=pltpu.PrefetchScalarGridSpec(
            num_scalar_prefetch=1,
            grid=grid,
            in_specs=[
                pl.BlockSpec(memory_space=pl.ANY),
                pl.BlockSpec((seq_tile, H), lambda g, *_: (g % n_seq, 0)),
                pl.BlockSpec((1, H), lambda g, *_: (0, 0)),
                pl.BlockSpec((1, H), lambda g, *_: (0, 0)),
            ],
            out_specs=pl.BlockSpec((seq_tile, H), lambda g, *_: (g, 0)),
            scratch_shapes=[
                pltpu.VMEM((seq_tile, H), jnp.float32),
                pltpu.SemaphoreType.DMA((n_waves,)),
            ]),
        compiler_params=pltpu.CompilerParams(
            dimension_semantics=("parallel",),
            disable_bounds_checks=True,
            vmem_limit_bytes=64 << 20),
    )(ids.reshape(-1), word_table.astype(jnp.float32), pos, gamma2, beta2)

    out = out.reshape(B, s_pad, H)
    return out if s_pad == S else out[:, :S, :]


# speedup vs baseline: 5.3406x; 1.0020x over previous
import functools

import jax
import jax.numpy as jnp
from jax.experimental import pallas as pl
from jax.experimental.pallas import tpu as pltpu

_EPS = 1e-5
_SEQ_TILE_MAX = 256


def _round_up(x: int, m: int) -> int:
    return (x + m - 1) // m * m


def _gather_ln_kernel(seq_tile, n_waves,
                      ids_ref,
                      word_hbm,
                      pos_ref,
                      gamma_ref,
                      beta_ref,
                      out_ref,
                      tok_buf,
                      sems):
    g = pl.program_id(0) * pl.num_programs(1) + pl.program_id(1)
    base = g * seq_tile
    wave = seq_tile // n_waves

    for i in range(seq_tile):
        rid = ids_ref[base + i]
        pltpu.make_async_copy(word_hbm.at[pl.ds(rid, 1)],
                              tok_buf.at[pl.ds(i, 1)],
                              sems.at[i // wave]).start(priority=i & 1)

    gamma = gamma_ref[...]
    beta = beta_ref[...]

    for w in range(n_waves):
        rows = pl.ds(w * wave, wave)
        pltpu.make_async_copy(word_hbm.at[pl.ds(0, wave)],
                              tok_buf.at[rows],
                              sems.at[w]).wait()
        z = tok_buf[rows, :] + pos_ref[rows, :]
        mean = jnp.mean(z, axis=-1, keepdims=True)
        c = z - mean
        var = jnp.mean(c * c, axis=-1, keepdims=True)
        out_ref[rows, :] = c * jax.lax.rsqrt(var + _EPS) * gamma + beta


def kernel(input_ids, word_table, pos_table, gamma, beta):
    B, S = input_ids.shape
    V, H = word_table.shape

    seq_tile = min(_round_up(_SEQ_TILE_MAX, 8), _round_up(S, 8))
    s_pad = _round_up(S, seq_tile)
    n_seq = s_pad // seq_tile
    n_waves = 4 if seq_tile % 4 == 0 and (seq_tile // 4) % 8 == 0 else 1

    ids = jnp.clip(input_ids.astype(jnp.int32), 0, V - 1)
    if s_pad != S:
        ids = jnp.pad(ids, ((0, 0), (0, s_pad - S)))
    pos = pos_table[:S].astype(jnp.float32)
    if s_pad != S:
        pos = jnp.pad(pos, ((0, s_pad - S), (0, 0)))

    gamma2 = gamma.reshape(1, H).astype(jnp.float32)
    beta2 = beta.reshape(1, H).astype(jnp.float32)

    n_tiles = B * n_seq
    n_cores = 2 if n_tiles % 2 == 0 else 1
    grid = (n_cores, n_tiles // n_cores)
    kernel_fn = functools.partial(_gather_ln_kernel, seq_tile, n_waves)
    out = pl.pallas_call(
        kernel_fn,
        out_shape=jax.ShapeDtypeStruct((B * s_pad, H), jnp.float32),
        grid_spec=pltpu.PrefetchScalarGridSpec(
            num_scalar_prefetch=1,
            grid=grid,
            in_specs=[
                pl.BlockSpec(memory_space=pl.ANY),
                pl.BlockSpec((seq_tile, H),
                             lambda c, t, *_: ((c * (n_tiles // n_cores) + t) % n_seq, 0)),
                pl.BlockSpec((1, H), lambda c, t, *_: (0, 0)),
                pl.BlockSpec((1, H), lambda c, t, *_: (0, 0)),
            ],
            out_specs=pl.BlockSpec((seq_tile, H),
                                   lambda c, t, *_: (c * (n_tiles // n_cores) + t, 0)),
            scratch_shapes=[
                pltpu.VMEM((seq_tile, H), jnp.float32),
                pltpu.SemaphoreType.DMA((n_waves,)),
            ]),
        compiler_params=pltpu.CompilerParams(
            dimension_semantics=("parallel", "arbitrary"),
            disable_bounds_checks=True,
            vmem_limit_bytes=64 << 20),
    )(ids.reshape(-1), word_table.astype(jnp.float32), pos, gamma2, beta2)

    out = out.reshape(B, s_pad, H)
    return out if s_pad == S else out[:, :S, :]


# speedup vs baseline: 6.0565x; 1.1341x over previous
import functools

import jax
import jax.numpy as jnp
from jax.experimental import pallas as pl
from jax.experimental.pallas import tpu as pltpu

_EPS = 1e-5
_TILE_ROWS = 512
_WAVE = 64


def _round_up(x: int, m: int) -> int:
    return (x + m - 1) // m * m


def _gather_ln_kernel(tile, n_waves,
                      ids_ref,
                      word_hbm,
                      pos_ref,
                      gamma_ref,
                      beta_ref,
                      out_ref,
                      tok_buf,
                      sems):
    g = pl.program_id(0) * pl.num_programs(1) + pl.program_id(1)
    base = g * tile
    wave = tile // n_waves

    for i in range(tile):
        rid = ids_ref[base + i]
        pltpu.make_async_copy(word_hbm.at[pl.ds(rid, 1)],
                              tok_buf.at[pl.ds(i, 1)],
                              sems.at[i // wave]).start()

    gamma = gamma_ref[...]
    beta = beta_ref[...]

    for w in range(n_waves):
        rows = pl.ds(w * wave, wave)
        pltpu.make_async_copy(word_hbm.at[pl.ds(0, wave)],
                              tok_buf.at[rows],
                              sems.at[w]).wait()
        z = tok_buf[rows, :] + pos_ref[rows, :]
        mean = jnp.mean(z, axis=-1, keepdims=True)
        c = z - mean
        var = jnp.mean(c * c, axis=-1, keepdims=True)
        out_ref[rows, :] = c * jax.lax.rsqrt(var + _EPS) * gamma + beta


def kernel(input_ids, word_table, pos_table, gamma, beta):
    B, S = input_ids.shape
    V, H = word_table.shape

    s_pad = _round_up(S, 8)
    n_rows = B * s_pad
    tile = _TILE_ROWS
    while n_rows % tile != 0:
        tile //= 2
    n_tiles = n_rows // tile
    n_waves = max(1, tile // _WAVE)

    ids = jnp.clip(input_ids.astype(jnp.int32), 0, V - 1)
    if s_pad != S:
        ids = jnp.pad(ids, ((0, 0), (0, s_pad - S)))
    pos = pos_table[:S].astype(jnp.float32)
    if s_pad != S:
        pos = jnp.pad(pos, ((0, s_pad - S), (0, 0)))

    if tile % s_pad == 0:
        pos_arr = jnp.tile(pos, (tile // s_pad, 1))
        n_pos_blocks = 1
    else:
        assert s_pad % tile == 0
        pos_arr = pos
        n_pos_blocks = s_pad // tile

    gamma2 = gamma.reshape(1, H).astype(jnp.float32)
    beta2 = beta.reshape(1, H).astype(jnp.float32)

    n_cores = 2 if n_tiles % 2 == 0 else 1
    tiles_per_core = n_tiles // n_cores
    grid = (n_cores, tiles_per_core)

    def _tile_idx(c, t):
        return c * tiles_per_core + t

    kernel_fn = functools.partial(_gather_ln_kernel, tile, n_waves)
    out = pl.pallas_call(
        kernel_fn,
        out_shape=jax.ShapeDtypeStruct((n_rows, H), jnp.float32),
        grid_spec=pltpu.PrefetchScalarGridSpec(
            num_scalar_prefetch=1,
            grid=grid,
            in_specs=[
                pl.BlockSpec(memory_space=pl.ANY),
                pl.BlockSpec((tile, H),
                             lambda c, t, *_: (_tile_idx(c, t) % n_pos_blocks, 0)),
                pl.BlockSpec((1, H), lambda c, t, *_: (0, 0)),
                pl.BlockSpec((1, H), lambda c, t, *_: (0, 0)),
            ],
            out_specs=pl.BlockSpec((tile, H),
                                   lambda c, t, *_: (_tile_idx(c, t), 0)),
            scratch_shapes=[
                pltpu.VMEM((tile, H), jnp.float32),
                pltpu.SemaphoreType.DMA((n_waves,)),
            ]),
        compiler_params=pltpu.CompilerParams(
            dimension_semantics=("parallel", "arbitrary"),
            disable_bounds_checks=True,
            vmem_limit_bytes=64 << 20),
    )(ids.reshape(-1), word_table.astype(jnp.float32), pos_arr, gamma2, beta2)

    out = out.reshape(B, s_pad, H)
    return out if s_pad == S else out[:, :S, :]
